# baseline (speedup 1.0000x reference)
import functools

import jax
import jax.numpy as jnp
from jax import lax
from jax.experimental import pallas as pl
from jax.experimental.pallas import tpu as pltpu
from jax.experimental.pallas import tpu_sc as plsc

VOCAB = 1000000
EMBED_DIM = 64
OUTPUT_DIM = 16
SEQ = 200
BATCH = 4096

NC = 2
NS = 16
NW = NC * NS
BPW = BATCH // NW
LANES = 16
DGRP = EMBED_DIM // LANES

T = 5
NBUF = 2
CHUNKS = SEQ // T
CROWS = T * BPW


def _pertile_kernel(x_ref, o_ref):
    for w in range(NW):
        o_ref[w] = x_ref[:, pl.ds(w * BPW, BPW)]


@jax.jit
def _pertile(text):
    return pl.pallas_call(
        _pertile_kernel,
        out_shape=jax.ShapeDtypeStruct((NW, SEQ, BPW), jnp.int32),
    )(text)


def _fasttext_kernel(textw, table, wt, bias, out, idx_v, idx1_v, rows_v,
                     acc_v, w_v, b_v, outb_v, *sems):
    wid = lax.axis_index("s") * NC + lax.axis_index("c")
    base = wid * BPW

    pltpu.sync_copy(textw.at[wid], idx_v)
    pltpu.sync_copy(wt, w_v)
    pltpu.sync_copy(bias, b_v)

    zero = jnp.zeros((LANES,), jnp.float32)

    def zacc(b, carry):
        for g in range(DGRP):
            acc_v[b, pl.ds(g * LANES, LANES)] = zero
        return carry

    lax.fori_loop(0, BPW, zacc, 0, unroll=8)

    def start_gather(c, db):
        for r in range(T):
            for g in range(BPW // LANES):
                idx1_v[db, pl.ds(r * BPW + g * LANES, LANES)] = (
                    idx_v[c * T + r, pl.ds(g * LANES, LANES)])
        pltpu.make_async_copy(
            table.at[idx1_v.at[db]], rows_v.at[db], sems[db],
        ).start()

    def wait_gather(db):
        pltpu.make_async_copy(
            table.at[idx1_v.at[db]], rows_v.at[db], sems[db],
        ).wait()

    def accumulate(db):
        for tt in range(T):
            def body(b, carry):
                for g in range(DGRP):
                    v = rows_v[db, tt * BPW + b, pl.ds(g * LANES, LANES)]
                    plsc.addupdate(acc_v.at[b, pl.ds(g * LANES, LANES)], v)
                return carry

            lax.fori_loop(0, BPW, body, 0, unroll=4)

    for j in range(NBUF - 1):
        start_gather(j, j)

    def group(q, carry):
        for j in range(NBUF):
            c = NBUF * q + j

            @pl.when(c + NBUF - 1 < CHUNKS)
            def _():
                start_gather(c + NBUF - 1, (j + NBUF - 1) % NBUF)

            wait_gather(j)
            accumulate(j)
        return carry

    lax.fori_loop(0, CHUNKS // NBUF, group, 0)

    b_row = b_v[:]

    def fc(b, carry):
        accs = [acc_v[b, pl.ds(g * LANES, LANES)] for g in range(DGRP)]
        o_v = zero
        for d in range(EMBED_DIM):
            o_v = o_v + accs[d // LANES][d % LANES] * w_v[d]
        outb_v[b] = o_v * (1.0 / SEQ) + b_row
        return carry

    lax.fori_loop(0, BPW, fc, 0)

    pltpu.sync_copy(outb_v, out.at[pl.ds(base, BPW)])


@jax.jit
def _fasttext(textw, table, wt, bias):
    mesh = plsc.VectorSubcoreMesh(
        core_axis_name="c", subcore_axis_name="s", num_cores=NC,
        num_subcores=NS)
    return pl.kernel(
        _fasttext_kernel,
        out_type=jax.ShapeDtypeStruct((BATCH, OUTPUT_DIM), jnp.float32),
        mesh=mesh,
        compiler_params=pltpu.CompilerParams(use_tc_tiling_on_sc=False),
        scratch_types=[
            pltpu.VMEM((SEQ, BPW), jnp.int32),
            pltpu.VMEM((NBUF, CROWS), jnp.int32),
            pltpu.VMEM((NBUF, CROWS, EMBED_DIM), jnp.float32),
            pltpu.VMEM((BPW, EMBED_DIM), jnp.float32),
            pltpu.VMEM((EMBED_DIM, OUTPUT_DIM), jnp.float32),
            pltpu.VMEM((OUTPUT_DIM,), jnp.float32),
            pltpu.VMEM((BPW, OUTPUT_DIM), jnp.float32),
        ] + [pltpu.SemaphoreType.DMA] * NBUF,
    )(textw, table, wt, bias)


def kernel(text, emb_table, fc_w, fc_b):
    textw = _pertile(text.astype(jnp.int32))
    return _fasttext(textw, emb_table,
                     fc_w.T.astype(jnp.float32), fc_b.astype(jnp.float32))


# speedup vs baseline: 1.7785x; 1.7785x over previous
import functools

import jax
import jax.numpy as jnp
from jax import lax
from jax.experimental import pallas as pl
from jax.experimental.pallas import tpu as pltpu
from jax.experimental.pallas import tpu_sc as plsc

VOCAB = 1000000
EMBED_DIM = 64
OUTPUT_DIM = 16
SEQ = 200
BATCH = 4096

NC = 2
NS = 16
NW = NC * NS
BPW = BATCH // NW
LANES = 16
DGRP = EMBED_DIM // LANES

T = 5
NBUF = 2
CHUNKS = SEQ // T
CROWS = T * BPW


VCP = 4096
GPACK = -(-VOCAB // (2 * VCP))
VROWS = GPACK * VCP * 2


def _packtable_kernel(x_ref, o_ref):
    o_ref[:, pl.ds(0, EMBED_DIM)] = jnp.transpose(x_ref[:, pl.ds(0, VCP)])
    o_ref[:, pl.ds(EMBED_DIM, EMBED_DIM)] = jnp.transpose(
        x_ref[:, pl.ds(VCP, VCP)])


@jax.jit
def _packtable(table_t):
    return pl.pallas_call(
        _packtable_kernel,
        grid=(GPACK,),
        in_specs=[pl.BlockSpec((EMBED_DIM, 2 * VCP), lambda c: (0, c))],
        out_specs=pl.BlockSpec((VCP, 2 * EMBED_DIM), lambda c: (c, 0)),
        out_shape=jax.ShapeDtypeStruct((VROWS // 2, 2 * EMBED_DIM),
                                       jnp.float32),
    )(table_t)


def _pertile_kernel(x_ref, o_ref):
    for w in range(NW):
        o_ref[w] = x_ref[:, pl.ds(w * BPW, BPW)]


@jax.jit
def _pertile(text):
    return pl.pallas_call(
        _pertile_kernel,
        out_shape=jax.ShapeDtypeStruct((NW, SEQ, BPW), jnp.int32),
    )(text)


def _fasttext_kernel(textw, table, wt, bias, out, idx_v, idx1_v, rows_v,
                     acc_v, w_v, b_v, outb_v, *sems):
    wid = lax.axis_index("s") * NC + lax.axis_index("c")
    base = wid * BPW

    pltpu.sync_copy(textw.at[wid], idx_v)
    pltpu.sync_copy(wt, w_v)
    pltpu.sync_copy(bias, b_v)

    zero = jnp.zeros((LANES,), jnp.float32)

    def zacc(b, carry):
        for g in range(DGRP):
            acc_v[b, pl.ds(g * LANES, LANES)] = zero
        return carry

    lax.fori_loop(0, BPW, zacc, 0, unroll=8)

    def start_gather(c, db):
        for r in range(T):
            for g in range(BPW // LANES):
                v = idx_v[c * T + r, pl.ds(g * LANES, LANES)]
                v2 = (((v >> 13) << 13) + ((v & 4095) << 1)
                      + ((v >> 12) & 1))
                idx1_v[db, pl.ds(r * BPW + g * LANES, LANES)] = v2
        pltpu.make_async_copy(
            table.at[idx1_v.at[db]], rows_v.at[db], sems[db],
        ).start()

    def wait_gather(db):
        pltpu.make_async_copy(
            table.at[idx1_v.at[db]], rows_v.at[db], sems[db],
        ).wait()

    def accumulate(db):
        for tt in range(T):
            def body(b, carry):
                for g in range(DGRP):
                    v = rows_v[db, tt * BPW + b, pl.ds(g * LANES, LANES)]
                    plsc.addupdate(acc_v.at[b, pl.ds(g * LANES, LANES)], v)
                return carry

            lax.fori_loop(0, BPW, body, 0, unroll=4)

    for j in range(NBUF - 1):
        start_gather(j, j)

    def group(q, carry):
        for j in range(NBUF):
            c = NBUF * q + j

            @pl.when(c + NBUF - 1 < CHUNKS)
            def _():
                start_gather(c + NBUF - 1, (j + NBUF - 1) % NBUF)

            wait_gather(j)
            accumulate(j)
        return carry

    lax.fori_loop(0, CHUNKS // NBUF, group, 0)

    b_row = b_v[:]

    def fc(b, carry):
        accs = [acc_v[b, pl.ds(g * LANES, LANES)] for g in range(DGRP)]
        o_v = zero
        for d in range(EMBED_DIM):
            o_v = o_v + accs[d // LANES][d % LANES] * w_v[d]
        outb_v[b] = o_v * (1.0 / SEQ) + b_row
        return carry

    lax.fori_loop(0, BPW, fc, 0)

    pltpu.sync_copy(outb_v, out.at[pl.ds(base, BPW)])


@jax.jit
def _fasttext(textw, table, wt, bias):
    mesh = plsc.VectorSubcoreMesh(
        core_axis_name="c", subcore_axis_name="s", num_cores=NC,
        num_subcores=NS)
    return pl.kernel(
        _fasttext_kernel,
        out_type=jax.ShapeDtypeStruct((BATCH, OUTPUT_DIM), jnp.float32),
        mesh=mesh,
        compiler_params=pltpu.CompilerParams(use_tc_tiling_on_sc=False),
        scratch_types=[
            pltpu.VMEM((SEQ, BPW), jnp.int32),
            pltpu.VMEM((NBUF, CROWS), jnp.int32),
            pltpu.VMEM((NBUF, CROWS, EMBED_DIM), jnp.float32),
            pltpu.VMEM((BPW, EMBED_DIM), jnp.float32),
            pltpu.VMEM((EMBED_DIM, OUTPUT_DIM), jnp.float32),
            pltpu.VMEM((OUTPUT_DIM,), jnp.float32),
            pltpu.VMEM((BPW, OUTPUT_DIM), jnp.float32),
        ] + [pltpu.SemaphoreType.DMA] * NBUF,
    )(textw, table, wt, bias)


def kernel(text, emb_table, fc_w, fc_b):
    textw = _pertile(text.astype(jnp.int32))
    packed = _packtable(emb_table.T)
    table = packed.reshape(VROWS, EMBED_DIM)
    return _fasttext(textw, table,
                     fc_w.T.astype(jnp.float32), fc_b.astype(jnp.float32))


# speedup vs baseline: 2.2819x; 1.2831x over previous
import jax
import jax.numpy as jnp
from jax import lax
from jax.experimental import pallas as pl
from jax.experimental.pallas import tpu as pltpu
from jax.experimental.pallas import tpu_sc as plsc

VOCAB = 1000000
EMBED_DIM = 64
OUTPUT_DIM = 16
SEQ = 200
BATCH = 4096

NC = 2
NS = 16
NW = NC * NS
BPW = BATCH // NW
LANES = 16

T = 5
NBUF = 2
CHUNKS = SEQ // T
CROWS = T * BPW

OPL = 128 // OUTPUT_DIM
VCP = 4096
VBLK = OPL * VCP
GPROJ = -(-VOCAB // VBLK)


def _project_kernel(x_ref, w_ref, o_ref):
    w = w_ref[...]
    for g in range(OPL):
        x = x_ref[:, pl.ds(g * VCP, VCP)]
        o_ref[:, pl.ds(g * OUTPUT_DIM, OUTPUT_DIM)] = lax.dot_general(
            x, w, (((0,), (1,)), ((), ())),
            preferred_element_type=jnp.float32)


@jax.jit
def _project(table_t, fc_w):
    return pl.pallas_call(
        _project_kernel,
        grid=(GPROJ,),
        in_specs=[
            pl.BlockSpec((EMBED_DIM, VBLK), lambda c: (0, c)),
            pl.BlockSpec((OUTPUT_DIM, EMBED_DIM), lambda c: (0, 0)),
        ],
        out_specs=pl.BlockSpec((VCP, OPL * OUTPUT_DIM), lambda c: (c, 0)),
        out_shape=jax.ShapeDtypeStruct((GPROJ * VCP, OPL * OUTPUT_DIM),
                                       jnp.float32),
    )(table_t, fc_w)


def _pertile_kernel(x_ref, o_ref):
    for w in range(NW):
        o_ref[w] = x_ref[:, pl.ds(w * BPW, BPW)]


@jax.jit
def _pertile(text):
    return pl.pallas_call(
        _pertile_kernel,
        out_shape=jax.ShapeDtypeStruct((NW, SEQ, BPW), jnp.int32),
    )(text)


def _fasttext_kernel(textw, ptable, bias, out, idx_v, idx1_v, rows_v,
                     acc_v, b_v, outb_v, *sems):
    wid = lax.axis_index("s") * NC + lax.axis_index("c")
    base = wid * BPW

    pltpu.sync_copy(textw.at[wid], idx_v)
    pltpu.sync_copy(bias, b_v)

    zero = jnp.zeros((LANES,), jnp.float32)

    def zacc(b, carry):
        acc_v[b] = zero
        return carry

    lax.fori_loop(0, BPW, zacc, 0, unroll=8)

    def start_gather(c, db):
        for r in range(T):
            for g in range(BPW // LANES):
                v = idx_v[c * T + r, pl.ds(g * LANES, LANES)]
                v2 = (((v >> 15) << 15) + ((v & 4095) << 3)
                      + ((v >> 12) & 7))
                idx1_v[db, pl.ds(r * BPW + g * LANES, LANES)] = v2
        pltpu.make_async_copy(
            ptable.at[idx1_v.at[db]], rows_v.at[db], sems[db],
        ).start()

    def wait_gather(db):
        pltpu.make_async_copy(
            ptable.at[idx1_v.at[db]], rows_v.at[db], sems[db],
        ).wait()

    def accumulate(db):
        for tt in range(T):
            def body(b, carry):
                plsc.addupdate(acc_v.at[b], rows_v[db, tt * BPW + b])
                return carry

            lax.fori_loop(0, BPW, body, 0, unroll=8)

    for j in range(NBUF - 1):
        start_gather(j, j)

    def group(q, carry):
        for j in range(NBUF):
            c = NBUF * q + j

            @pl.when(c + NBUF - 1 < CHUNKS)
            def _():
                start_gather(c + NBUF - 1, (j + NBUF - 1) % NBUF)

            wait_gather(j)
            accumulate(j)
        return carry

    lax.fori_loop(0, CHUNKS // NBUF, group, 0)

    b_row = b_v[:]

    def fin(b, carry):
        outb_v[b] = acc_v[b] * (1.0 / SEQ) + b_row
        return carry

    lax.fori_loop(0, BPW, fin, 0, unroll=8)

    pltpu.sync_copy(outb_v, out.at[pl.ds(base, BPW)])


@jax.jit
def _fasttext(textw, ptable, bias):
    mesh = plsc.VectorSubcoreMesh(
        core_axis_name="c", subcore_axis_name="s", num_cores=NC,
        num_subcores=NS)
    return pl.kernel(
        _fasttext_kernel,
        out_type=jax.ShapeDtypeStruct((BATCH, OUTPUT_DIM), jnp.float32),
        mesh=mesh,
        compiler_params=pltpu.CompilerParams(use_tc_tiling_on_sc=False),
        scratch_types=[
            pltpu.VMEM((SEQ, BPW), jnp.int32),
            pltpu.VMEM((NBUF, CROWS), jnp.int32),
            pltpu.VMEM((NBUF, CROWS, OUTPUT_DIM), jnp.float32),
            pltpu.VMEM((BPW, OUTPUT_DIM), jnp.float32),
            pltpu.VMEM((OUTPUT_DIM,), jnp.float32),
            pltpu.VMEM((BPW, OUTPUT_DIM), jnp.float32),
        ] + [pltpu.SemaphoreType.DMA] * NBUF,
    )(textw, ptable, bias)


def kernel(text, emb_table, fc_w, fc_b):
    textw = _pertile(text.astype(jnp.int32))
    packed = _project(emb_table.T, fc_w.astype(jnp.float32))
    ptable = packed.reshape(GPROJ * VCP * OPL, OUTPUT_DIM)
    return _fasttext(textw, ptable, fc_b.astype(jnp.float32))
